# Initial kernel scaffold; baseline (speedup 1.0000x reference)
import jax
import jax.numpy as jnp
from jax.experimental import pallas as pl


def kernel(bboxes_cxcywh, conf):
    raise NotImplementedError("write your pallas kernel here")


# speedup vs baseline: 628.1569x; 628.1569x over previous
import jax
import jax.numpy as jnp
from jax import lax
from jax.experimental import pallas as pl
from jax.experimental.pallas import tpu as pltpu

N_BOX = 1000
N_PAD = 1024
INF = 1e9
IOU_T = 0.1


def _shift_lanes(v, k):
    return jnp.concatenate(
        [jnp.full((8, k), INF, jnp.float32), v[:, : 128 - k]], axis=1
    )


def _shift_subs(v, k):
    return jnp.concatenate(
        [jnp.full((k, 1), INF, jnp.float32), v[: 8 - k, :]], axis=0
    )


def _body(cv_ref, cs_ref, conf_ref, out_ref):
    cx, cy, w, h = cv_ref[0], cv_ref[1], cv_ref[2], cv_ref[3]
    X1 = cx - 0.5 * w
    Y1 = cy - 0.5 * h
    X2 = cx + 0.5 * w
    Y2 = cy + 0.5 * h
    AREA = w * h
    ROW = lax.broadcasted_iota(jnp.int32, (8, 128), 0)
    LANE = lax.broadcasted_iota(jnp.int32, (8, 128), 1)
    IDX = (ROW * 128 + LANE).astype(jnp.float32)
    CONF = conf_ref[:]

    def rowbody(i, assign):
        fi = i.astype(jnp.float32)
        ws = cs_ref[2, i]
        hs = cs_ref[3, i]
        x1s = cs_ref[0, i] - 0.5 * ws
        y1s = cs_ref[1, i] - 0.5 * hs
        x2s = x1s + ws
        y2s = y1s + hs
        iw = jnp.maximum(jnp.minimum(x2s, X2) - jnp.maximum(x1s, X1), 0.0)
        ih = jnp.maximum(jnp.minimum(y2s, Y2) - jnp.maximum(y1s, Y1), 0.0)
        inter = iw * ih
        union = ws * hs + AREA - inter
        maskj = (inter > IOU_T * union) & (IDX > fi)
        vfull = jnp.where(maskj | (IDX == fi), assign, INF)
        p = vfull
        for k in (1, 2, 4, 8, 16, 32, 64):
            p = jnp.minimum(p, _shift_lanes(p, k))
        rowtot = p[:, 127:128]
        t = _shift_subs(rowtot, 1)
        for k in (1, 2, 4):
            t = jnp.minimum(t, _shift_subs(t, k))
        p = jnp.minimum(p, t)
        tot = jnp.min(vfull)
        assign = jnp.where(maskj, p, assign)
        return jnp.where(IDX == fi, tot, assign)

    assign = lax.fori_loop(0, N_BOX, rowbody, IDX, unroll=False)

    def cbody(c, maskacc):
        fc = c.astype(jnp.float32)
        M = assign == fc
        cnt = jnp.sum(jnp.where(M, 1.0, 0.0))
        mc = jnp.max(jnp.where(M, CONF, -INF))
        g = jnp.min(jnp.where(M & (CONF == mc), IDX, INF))
        local = jnp.sum(jnp.where(M & (IDX < g), 1.0, 0.0))
        repr_ = jnp.where(cnt == 1.0, g, local)
        hit = (IDX == repr_) & (cnt > 0.0)
        return jnp.maximum(maskacc, jnp.where(hit, 1.0, 0.0))

    maskacc = lax.fori_loop(
        0, N_BOX, cbody, jnp.zeros((8, 128), jnp.float32), unroll=False
    )
    out_ref[:, :] = maskacc


def kernel(bboxes_cxcywh, conf):
    coords = jnp.transpose(bboxes_cxcywh).astype(jnp.float32)
    coords = jnp.pad(coords, ((0, 0), (0, N_PAD - N_BOX)))
    coords_vec = coords.reshape(4, 8, 128)
    confp = jnp.pad(conf.astype(jnp.float32), (0, N_PAD - N_BOX)).reshape(8, 128)
    out = pl.pallas_call(
        _body,
        in_specs=[
            pl.BlockSpec(memory_space=pltpu.VMEM),
            pl.BlockSpec(memory_space=pltpu.SMEM),
            pl.BlockSpec(memory_space=pltpu.VMEM),
        ],
        out_specs=pl.BlockSpec(memory_space=pltpu.VMEM),
        out_shape=jax.ShapeDtypeStruct((8, 128), jnp.float32),
    )(coords_vec, coords, confp)
    return out.reshape(N_PAD)[:N_BOX] > 0.5


# speedup vs baseline: 962.6099x; 1.5324x over previous
import jax
import jax.numpy as jnp
from jax import lax
from jax.experimental import pallas as pl
from jax.experimental.pallas import tpu as pltpu

N_BOX = 1000
N_PAD = 1024
INF = 1e9
IOU_T = 0.1


def _shift_lanes(v, k):
    return jnp.concatenate(
        [jnp.full((8, k), INF, jnp.float32), v[:, : 128 - k]], axis=1
    )


def _shift_subs(v, k):
    return jnp.concatenate(
        [jnp.full((k, 1), INF, jnp.float32), v[: 8 - k, :]], axis=0
    )


def _body(cv_ref, cs_ref, conf_ref, out_ref):
    cx, cy, w, h = cv_ref[0], cv_ref[1], cv_ref[2], cv_ref[3]
    X1 = cx - 0.5 * w
    Y1 = cy - 0.5 * h
    X2 = cx + 0.5 * w
    Y2 = cy + 0.5 * h
    AREA = w * h
    ROW = lax.broadcasted_iota(jnp.int32, (8, 128), 0)
    LANE = lax.broadcasted_iota(jnp.int32, (8, 128), 1)
    IDX = (ROW * 128 + LANE).astype(jnp.float32)
    CONF = conf_ref[:]

    def rowbody(i, assign):
        fi = i.astype(jnp.float32)
        ws = cs_ref[2, i]
        hs = cs_ref[3, i]
        x1s = cs_ref[0, i] - 0.5 * ws
        y1s = cs_ref[1, i] - 0.5 * hs
        x2s = x1s + ws
        y2s = y1s + hs
        iw = jnp.maximum(jnp.minimum(x2s, X2) - jnp.maximum(x1s, X1), 0.0)
        ih = jnp.maximum(jnp.minimum(y2s, Y2) - jnp.maximum(y1s, Y1), 0.0)
        inter = iw * ih
        union = ws * hs + AREA - inter
        maskj = (inter > IOU_T * union) & (IDX > fi)
        vfull = jnp.where(maskj | (IDX == fi), assign, INF)
        p = vfull
        for k in (1, 2, 4, 8, 16, 32, 64):
            p = jnp.minimum(p, _shift_lanes(p, k))
        rowtot = p[:, 127:128]
        t = _shift_subs(rowtot, 1)
        for k in (1, 2, 4):
            t = jnp.minimum(t, _shift_subs(t, k))
        p = jnp.minimum(p, t)
        tot = jnp.min(vfull)
        assign = jnp.where(maskj, p, assign)
        return jnp.where(IDX == fi, tot, assign)

    assign = lax.fori_loop(0, N_BOX, rowbody, IDX, unroll=False)

    def _r3(x, op):
        return op(op(x, axis=2, keepdims=True), axis=1, keepdims=True)

    def cblock(cb, maskacc):
        c0 = (cb * 128).astype(jnp.float32)
        C = lax.broadcasted_iota(jnp.int32, (128, 1, 1), 0).astype(jnp.float32) + c0
        M = assign[None, :, :] == C
        cnt = _r3(jnp.where(M, 1.0, 0.0), jnp.sum)
        mc = _r3(jnp.where(M, CONF[None], -INF), jnp.max)
        g = _r3(jnp.where(M & (CONF[None] == mc), IDX[None], INF), jnp.min)
        loc = _r3(jnp.where(M & (IDX[None] < g), 1.0, 0.0), jnp.sum)
        repr_ = jnp.where(cnt == 1.0, g, loc)
        valid = (cnt > 0.0) & (C < float(N_BOX))
        hits = (IDX[None] == repr_) & valid
        contrib = jnp.max(jnp.where(hits, 1.0, 0.0), axis=0)
        return jnp.maximum(maskacc, contrib)

    maskacc = lax.fori_loop(0, 8, cblock, jnp.zeros((8, 128), jnp.float32))
    out_ref[:, :] = maskacc


def kernel(bboxes_cxcywh, conf):
    coords = jnp.transpose(bboxes_cxcywh).astype(jnp.float32)
    coords = jnp.pad(coords, ((0, 0), (0, N_PAD - N_BOX)))
    coords_vec = coords.reshape(4, 8, 128)
    confp = jnp.pad(conf.astype(jnp.float32), (0, N_PAD - N_BOX)).reshape(8, 128)
    out = pl.pallas_call(
        _body,
        in_specs=[
            pl.BlockSpec(memory_space=pltpu.VMEM),
            pl.BlockSpec(memory_space=pltpu.SMEM),
            pl.BlockSpec(memory_space=pltpu.VMEM),
        ],
        out_specs=pl.BlockSpec(memory_space=pltpu.VMEM),
        out_shape=jax.ShapeDtypeStruct((8, 128), jnp.float32),
    )(coords_vec, coords, confp)
    return out.reshape(N_PAD)[:N_BOX] > 0.5
